# Initial kernel scaffold; baseline (speedup 1.0000x reference)
import jax
import jax.numpy as jnp
from jax.experimental import pallas as pl


def kernel(x, edge_index, batch_index, W1, b1, attn, W_out, b_out):
    raise NotImplementedError("write your pallas kernel here")


# speedup vs baseline: 21.2504x; 21.2504x over previous
import functools

import jax
import jax.numpy as jnp
from jax import lax
from jax.experimental import pallas as pl
from jax.experimental.pallas import tpu as pltpu
from jax.experimental.pallas import tpu_sc as plsc

N = 10000
E = 320000
FEAT = 128
EMB = 32
G = 64
NC, NS = 2, 16
NW = NC * NS
PADN = 10240
ROWS_PER_TILE = PADN // NS
HB = 128
NBLK_TOTAL = E // HB
NBLK_BASE = NBLK_TOTAL // NW
NBLK_REM = NBLK_TOTAL % NW

def _sc_hist_body(dst_hbm, ones_hbm, zcol_hbm, out_hbm, idx_v, ones_v, zbuf, acc):
    c = lax.axis_index("c")
    s = lax.axis_index("s")
    w = c * NS + s
    pltpu.sync_copy(zcol_hbm.at[pl.ds(s * ROWS_PER_TILE, ROWS_PER_TILE)], zbuf)
    pltpu.sync_copy(zbuf, acc.at[pl.ds(s * ROWS_PER_TILE, ROWS_PER_TILE)])
    pltpu.sync_copy(ones_hbm, ones_v)
    plsc.subcore_barrier()

    nblk = NBLK_BASE + jnp.where(w < NBLK_REM, 1, 0)

    def blk(j, carry):
        off = (w + NW * j) * HB
        pltpu.sync_copy(dst_hbm.at[pl.ds(off, HB)], idx_v)
        pltpu.sync_copy(ones_v, acc.at[idx_v], add=True)
        return carry

    lax.fori_loop(0, nblk, blk, 0)
    plsc.subcore_barrier()

    @pl.when(s == 0)
    def _():
        pltpu.sync_copy(acc, out_hbm.at[c])


def _sc_agg_body(g_hbm, src_hbm, dst_hbm, zrows_hbm, out_hbm,
                 idxs_v, idxd_v, rows_v, zbuf, acc, sem):
    c = lax.axis_index("c")
    s = lax.axis_index("s")
    w = c * NS + s
    pltpu.sync_copy(zrows_hbm.at[pl.ds(s * ROWS_PER_TILE, ROWS_PER_TILE)], zbuf)
    pltpu.sync_copy(zbuf, acc.at[pl.ds(s * ROWS_PER_TILE, ROWS_PER_TILE)])
    plsc.subcore_barrier()

    nblk = NBLK_BASE + jnp.where(w < NBLK_REM, 1, 0)

    def blk(j, carry):
        off = (w + NW * j) * HB
        pltpu.sync_copy(src_hbm.at[pl.ds(off, HB)], idxs_v)
        pltpu.sync_copy(dst_hbm.at[pl.ds(off, HB)], idxd_v)
        pltpu.async_copy(g_hbm.at[idxs_v], rows_v, sem).wait()
        pltpu.sync_copy(rows_v, acc.at[idxd_v], add=True)
        return carry

    lax.fori_loop(0, nblk, blk, 0)
    plsc.subcore_barrier()
    pltpu.sync_copy(acc.at[pl.ds(s * ROWS_PER_TILE, ROWS_PER_TILE)], out_hbm.at[w])


@functools.lru_cache(maxsize=1)
def _sc_kernels():
    mesh = plsc.VectorSubcoreMesh(core_axis_name="c", subcore_axis_name="s",
                                  num_cores=NC, num_subcores=NS)
    params = pltpu.CompilerParams(use_tc_tiling_on_sc=False)
    sc_hist = pl.kernel(
        _sc_hist_body,
        out_type=jax.ShapeDtypeStruct((NC, PADN, 1), jnp.float32),
        mesh=mesh,
        compiler_params=params,
        scratch_types=[
            pltpu.VMEM((HB,), jnp.int32),
            pltpu.VMEM((HB, 1), jnp.float32),
            pltpu.VMEM((ROWS_PER_TILE, 1), jnp.float32),
            pltpu.VMEM_SHARED((PADN, 1), jnp.float32),
        ],
    )
    sc_agg = pl.kernel(
        _sc_agg_body,
        out_type=jax.ShapeDtypeStruct((NW, ROWS_PER_TILE, EMB), jnp.float32),
        mesh=mesh,
        compiler_params=params,
        scratch_types=[
            pltpu.VMEM((HB,), jnp.int32),
            pltpu.VMEM((HB,), jnp.int32),
            pltpu.VMEM((HB, EMB), jnp.float32),
            pltpu.VMEM((ROWS_PER_TILE, EMB), jnp.float32),
            pltpu.VMEM_SHARED((PADN, EMB), jnp.float32),
            pltpu.SemaphoreType.DMA,
        ],
    )
    return sc_hist, sc_agg


def _tca_body(hist_ref, x_ref, w1_ref, g_ref, dinv_ref):
    hist = hist_ref[...]
    deg = hist[0] + hist[1] + 1.0
    dinv = lax.rsqrt(deg)[:, None]
    h = jnp.dot(x_ref[...], w1_ref[...], preferred_element_type=jnp.float32)
    g_ref[...] = h * dinv
    dinv_ref[...] = dinv


_tca = pl.pallas_call(
    _tca_body,
    out_shape=[
        jax.ShapeDtypeStruct((N, EMB), jnp.float32),
        jax.ShapeDtypeStruct((N, 1), jnp.float32),
    ],
)


def _tcb_body(agg_ref, g_ref, dinv_ref, batch_ref, b1_ref, attn_ref,
              wout_ref, bout_ref, out_ref):
    agg = agg_ref[...]
    g = g_ref[...]
    dinv = dinv_ref[...]
    total = agg[0] + agg[1] + g
    out = jnp.maximum(dinv * total + b1_ref[...], 0.0)

    attn = attn_ref[...]
    nrm = jnp.sqrt(jnp.sum(attn * attn))
    score = jnp.dot(out, attn, preferred_element_type=jnp.float32) / nrm
    v = jnp.dot(out, wout_ref[...], preferred_element_type=jnp.float32)
    u = v * jnp.maximum(jnp.tanh(score), 0.0)

    ui = lax.bitcast_convert_type(score, jnp.uint32)
    key = jnp.where(ui >= jnp.uint32(0x80000000), ~ui,
                    ui | jnp.uint32(0x80000000))

    gid = lax.broadcasted_iota(jnp.int32, (1, G), 1)
    bm = batch_ref[...] == gid
    fone = jnp.float32(1.0)
    counts = jnp.sum(jnp.where(bm, fone, 0.0), axis=0, keepdims=True)
    k = jnp.ceil(jnp.float32(0.8) * counts)
    denom = jnp.maximum(k, 1.0)

    def _sel_bit(i, T):
        cand = T | (jnp.uint32(0x80000000) >> i)
        pred = bm & (key >= cand)
        cnt = jnp.sum(jnp.where(pred, fone, 0.0), axis=0, keepdims=True)
        return jnp.where(cnt >= k, cand, T)

    T = lax.fori_loop(0, 32, _sel_bit, jnp.zeros((1, G), jnp.uint32))

    strictly = bm & (key > T)
    cnt_gt = jnp.sum(jnp.where(strictly, fone, 0.0), axis=0, keepdims=True)
    m = k - cnt_gt
    tie = bm & (key == T)
    nidx = lax.broadcasted_iota(jnp.uint32, (N, 1), 0)

    def _tie_bit(i, V):
        cand = V | (jnp.uint32(1 << 13) >> i)
        c = jnp.sum(jnp.where(tie & (nidx < cand), fone, 0.0),
                    axis=0, keepdims=True)
        return jnp.where(c < m, cand, V)

    V = lax.fori_loop(0, 14, _tie_bit, jnp.zeros((1, G), jnp.uint32))

    sel = strictly | (tie & (nidx <= V))
    ssum = jnp.sum(jnp.where(sel, u, 0.0), axis=0, keepdims=True)
    out_ref[...] = ssum / denom + bout_ref[...]


_tcb = pl.pallas_call(
    _tcb_body,
    out_shape=jax.ShapeDtypeStruct((1, G), jnp.float32),
)


def kernel(x, edge_index, batch_index, W1, b1, attn, W_out, b_out):
    src = edge_index[0]
    dst = edge_index[1]
    ones_col = jnp.ones((HB, 1), jnp.float32)
    zcol = jnp.zeros((PADN, 1), jnp.float32)
    zrows = jnp.zeros((PADN, EMB), jnp.float32)

    sc_hist, sc_agg = _sc_kernels()
    histp = sc_hist(dst, ones_col, zcol)
    hist2 = histp.reshape(NC, PADN)[:, :N]
    g, dinv = _tca(hist2, x, W1)
    aggp = sc_agg(g, src, dst, zrows)
    agg2 = aggp.reshape(NC, PADN, EMB)[:, :N]
    ans = _tcb(agg2, g, dinv, batch_index.reshape(N, 1).astype(jnp.int32),
               b1.reshape(1, EMB), attn.reshape(EMB, 1), W_out,
               b_out.reshape(1, 1))
    return ans.reshape(G, 1)


# speedup vs baseline: 25.1339x; 1.1827x over previous
import functools

import jax
import jax.numpy as jnp
from jax import lax
from jax.experimental import pallas as pl
from jax.experimental.pallas import tpu as pltpu
from jax.experimental.pallas import tpu_sc as plsc

N = 10000
E = 320000
FEAT = 128
EMB = 32
G = 64
NC, NS = 2, 16
NW = NC * NS
PADN = 10240
ROWS_PER_TILE = PADN // NS
HB = 128
NBLK_TOTAL = E // HB
NBLK_BASE = NBLK_TOTAL // NW
NBLK_REM = NBLK_TOTAL % NW

DEPTH = 3
NGRP = NBLK_BASE // DEPTH


def _sc_hist_body(dst_hbm, ones_hbm, zcol_hbm, out_hbm,
                  idx0, idx1, idx2, ones_v, zbuf, acc, sem0, sem1, sem2):
    c = lax.axis_index("c")
    s = lax.axis_index("s")
    w = c * NS + s
    idxs = (idx0, idx1, idx2)
    sems = (sem0, sem1, sem2)
    pltpu.sync_copy(zcol_hbm.at[pl.ds(s * ROWS_PER_TILE, ROWS_PER_TILE)], zbuf)
    pltpu.sync_copy(zbuf, acc.at[pl.ds(s * ROWS_PER_TILE, ROWS_PER_TILE)])
    pltpu.sync_copy(ones_hbm, ones_v)
    plsc.subcore_barrier()

    def off(j):
        return (w + NW * j) * HB

    nblk = NBLK_BASE + jnp.where(w < NBLK_REM, 1, 0)

    def hblk(j, carry):
        pltpu.sync_copy(dst_hbm.at[pl.ds(off(j), HB)], idx0)
        pltpu.sync_copy(ones_v, acc.at[idx0], add=True)
        return carry

    lax.fori_loop(0, nblk, hblk, 0)
    plsc.subcore_barrier()

    @pl.when(s == 0)
    def _():
        pltpu.sync_copy(acc, out_hbm.at[c])


def _sc_agg_body(g_hbm, src_hbm, dst_hbm, zrows_hbm, out_hbm,
                 ia0, ia1, ia2, id0, id1, id2, r0, r1, r2,
                 zbuf, acc, sg0, sg1, sg2):
    c = lax.axis_index("c")
    s = lax.axis_index("s")
    w = c * NS + s
    ias = (ia0, ia1, ia2)
    ids = (id0, id1, id2)
    rows = (r0, r1, r2)
    sgs = (sg0, sg1, sg2)
    pltpu.sync_copy(zrows_hbm.at[pl.ds(s * ROWS_PER_TILE, ROWS_PER_TILE)], zbuf)
    pltpu.sync_copy(zbuf, acc.at[pl.ds(s * ROWS_PER_TILE, ROWS_PER_TILE)])
    plsc.subcore_barrier()

    def off(j):
        return (w + NW * j) * HB

    def start(j, p):
        pltpu.sync_copy(src_hbm.at[pl.ds(off(j), HB)], ias[p])
        pltpu.sync_copy(dst_hbm.at[pl.ds(off(j), HB)], ids[p])
        pltpu.async_copy(g_hbm.at[ias[p]], rows[p], sgs[p])

    def finish(j, p):
        pltpu.make_async_copy(g_hbm.at[ias[p]], rows[p], sgs[p]).wait()
        pltpu.sync_copy(rows[p], acc.at[ids[p]], add=True)

    for p in range(DEPTH):
        start(p, p)

    def group(gi, carry):
        for p in range(DEPTH):
            j = gi * DEPTH + p
            finish(j, p)
            start(j + DEPTH, p)
        return carry

    lax.fori_loop(0, NGRP - 1, group, 0)
    for p in range(DEPTH):
        finish((NGRP - 1) * DEPTH + p, p)

    def atail(j, carry):
        start(j, 0)
        finish(j, 0)
        return carry

    nblk = NBLK_BASE + jnp.where(w < NBLK_REM, 1, 0)
    lax.fori_loop(NBLK_BASE, nblk, atail, 0)

    plsc.subcore_barrier()
    pltpu.sync_copy(acc.at[pl.ds(s * ROWS_PER_TILE, ROWS_PER_TILE)], out_hbm.at[w])


@functools.lru_cache(maxsize=1)
def _sc_kernels():
    mesh = plsc.VectorSubcoreMesh(core_axis_name="c", subcore_axis_name="s",
                                  num_cores=NC, num_subcores=NS)
    params = pltpu.CompilerParams(use_tc_tiling_on_sc=False)
    sc_hist = pl.kernel(
        _sc_hist_body,
        out_type=jax.ShapeDtypeStruct((NC, PADN, 1), jnp.float32),
        mesh=mesh,
        compiler_params=params,
        scratch_types=[
            pltpu.VMEM((HB,), jnp.int32),
            pltpu.VMEM((HB,), jnp.int32),
            pltpu.VMEM((HB,), jnp.int32),
            pltpu.VMEM((HB, 1), jnp.float32),
            pltpu.VMEM((ROWS_PER_TILE, 1), jnp.float32),
            pltpu.VMEM_SHARED((PADN, 1), jnp.float32),
            pltpu.SemaphoreType.DMA,
            pltpu.SemaphoreType.DMA,
            pltpu.SemaphoreType.DMA,
        ],
    )
    sc_agg = pl.kernel(
        _sc_agg_body,
        out_type=jax.ShapeDtypeStruct((NW, ROWS_PER_TILE, EMB), jnp.float32),
        mesh=mesh,
        compiler_params=params,
        scratch_types=(
            [pltpu.VMEM((HB,), jnp.int32)] * 3
            + [pltpu.VMEM((HB,), jnp.int32)] * 3
            + [pltpu.VMEM((HB, EMB), jnp.float32)] * 3
            + [pltpu.VMEM((ROWS_PER_TILE, EMB), jnp.float32),
               pltpu.VMEM_SHARED((PADN, EMB), jnp.float32)]
            + [pltpu.SemaphoreType.DMA] * 3
        ),
    )
    return sc_hist, sc_agg


def _tcmm_body(x_ref, w1_ref, h_ref):
    h_ref[...] = jnp.dot(x_ref[...], w1_ref[...],
                         preferred_element_type=jnp.float32)


_tcmm = pl.pallas_call(
    _tcmm_body,
    out_shape=jax.ShapeDtypeStruct((N, EMB), jnp.float32),
)


def _tcscale_body(hist_ref, h_ref, g_ref, dinv_ref):
    hist = hist_ref[...]
    deg = hist[0] + hist[1] + 1.0
    dinv = lax.rsqrt(deg)[:, None]
    g_ref[...] = h_ref[...] * dinv
    dinv_ref[...] = dinv


_tcscale = pl.pallas_call(
    _tcscale_body,
    out_shape=[
        jax.ShapeDtypeStruct((N, EMB), jnp.float32),
        jax.ShapeDtypeStruct((N, 1), jnp.float32),
    ],
)


def _tcb_body(agg_ref, g_ref, dinv_ref, batch_ref, b1_ref, attn_ref,
              wout_ref, bout_ref, out_ref):
    agg = agg_ref[...]
    g = g_ref[...]
    dinv = dinv_ref[...]
    total = agg[0] + agg[1] + g
    out = jnp.maximum(dinv * total + b1_ref[...], 0.0)

    attn = attn_ref[...]
    nrm = jnp.sqrt(jnp.sum(attn * attn))
    score = jnp.dot(out, attn, preferred_element_type=jnp.float32) / nrm
    v = jnp.dot(out, wout_ref[...], preferred_element_type=jnp.float32)
    u = v * jnp.maximum(jnp.tanh(score), 0.0)

    ui = lax.bitcast_convert_type(score, jnp.uint32)
    key = jnp.where(ui >= jnp.uint32(0x80000000), ~ui,
                    ui | jnp.uint32(0x80000000))

    gid = lax.broadcasted_iota(jnp.int32, (1, G), 1)
    bm = batch_ref[...] == gid
    fone = jnp.float32(1.0)
    counts = jnp.sum(jnp.where(bm, fone, 0.0), axis=0, keepdims=True)
    k = jnp.ceil(jnp.float32(0.8) * counts)
    denom = jnp.maximum(k, 1.0)

    def _sel_bit(i, T):
        cand = T | (jnp.uint32(0x80000000) >> i)
        pred = bm & (key >= cand)
        cnt = jnp.sum(jnp.where(pred, fone, 0.0), axis=0, keepdims=True)
        return jnp.where(cnt >= k, cand, T)

    T = lax.fori_loop(0, 32, _sel_bit, jnp.zeros((1, G), jnp.uint32))

    strictly = bm & (key > T)
    cnt_gt = jnp.sum(jnp.where(strictly, fone, 0.0), axis=0, keepdims=True)
    m = k - cnt_gt
    tie = bm & (key == T)
    nidx = lax.broadcasted_iota(jnp.uint32, (N, 1), 0)

    def _tie_bit(i, V):
        cand = V | (jnp.uint32(1 << 13) >> i)
        c = jnp.sum(jnp.where(tie & (nidx < cand), fone, 0.0),
                    axis=0, keepdims=True)
        return jnp.where(c < m, cand, V)

    V = lax.fori_loop(0, 14, _tie_bit, jnp.zeros((1, G), jnp.uint32))

    sel = strictly | (tie & (nidx <= V))
    ssum = jnp.sum(jnp.where(sel, u, 0.0), axis=0, keepdims=True)
    out_ref[...] = ssum / denom + bout_ref[...]


_tcb = pl.pallas_call(
    _tcb_body,
    out_shape=jax.ShapeDtypeStruct((1, G), jnp.float32),
)


def kernel(x, edge_index, batch_index, W1, b1, attn, W_out, b_out):
    src = edge_index[0]
    dst = edge_index[1]
    ones_col = jnp.ones((HB, 1), jnp.float32)
    zcol = jnp.zeros((PADN, 1), jnp.float32)
    zrows = jnp.zeros((PADN, EMB), jnp.float32)

    sc_hist, sc_agg = _sc_kernels()
    h = _tcmm(x, W1)
    histp = sc_hist(dst, ones_col, zcol)
    hist2 = histp.reshape(NC, PADN)[:, :N]
    g, dinv = _tcscale(hist2, h)
    aggp = sc_agg(g, src, dst, zrows)
    agg2 = aggp.reshape(NC, PADN, EMB)[:, :N]
    ans = _tcb(agg2, g, dinv, batch_index.reshape(N, 1).astype(jnp.int32),
               b1.reshape(1, EMB), attn.reshape(EMB, 1), W_out,
               b_out.reshape(1, 1))
    return ans.reshape(G, 1)
